# baseline (speedup 1.0000x reference)
import jax
import jax.numpy as jnp
from jax.experimental import pallas as pl


_BLOCK = 256


def _bcast_kernel(table_ref, out_ref):
    for b in range(out_ref.shape[0]):
        out_ref[b] = table_ref[...]


def kernel(x, table):
    batch, seq_len = x.shape
    d_model = table.shape[1]
    n_blocks = seq_len // _BLOCK
    return pl.pallas_call(
        _bcast_kernel,
        grid=(n_blocks,),
        in_specs=[
            pl.BlockSpec((_BLOCK, d_model), lambda j: (j, 0)),
        ],
        out_specs=pl.BlockSpec((batch, _BLOCK, d_model), lambda j: (0, j, 0)),
        out_shape=jax.ShapeDtypeStruct((batch, seq_len, d_model), table.dtype),
    )(table)


# speedup vs baseline: 1.0949x; 1.0949x over previous
import jax
import jax.numpy as jnp
from jax.experimental import pallas as pl


_BLOCK = 1024


def _bcast_kernel(table_ref, out_ref):
    for b in range(out_ref.shape[0]):
        out_ref[b] = table_ref[...]


def kernel(x, table):
    batch, seq_len = x.shape
    d_model = table.shape[1]
    n_blocks = seq_len // _BLOCK
    return pl.pallas_call(
        _bcast_kernel,
        grid=(n_blocks,),
        in_specs=[
            pl.BlockSpec((_BLOCK, d_model), lambda j: (j, 0)),
        ],
        out_specs=pl.BlockSpec((batch, _BLOCK, d_model), lambda j: (0, j, 0)),
        out_shape=jax.ShapeDtypeStruct((batch, seq_len, d_model), table.dtype),
    )(table)
